# baseline (speedup 1.0000x reference)
import functools

import jax
import jax.numpy as jnp
from jax import lax
from jax.experimental import pallas as pl
from jax.experimental.pallas import tpu as pltpu
from jax.experimental.pallas import tpu_sc as plsc

NT = 1000000
BATCH = 16384

_info = plsc.get_sparse_core_info()
_NC, _NS = _info.num_cores, _info.num_subcores
_NW = _NC * _NS
_PER_W = BATCH // _NW
_CHUNK = 128
_N_CHUNK = _PER_W // _CHUNK

_mesh = plsc.VectorSubcoreMesh(core_axis_name="c", subcore_axis_name="s")


@functools.partial(
    pl.kernel,
    out_type=jax.ShapeDtypeStruct((BATCH,), jnp.float32),
    mesh=_mesh,
    scratch_types=[
        pltpu.VMEM((_PER_W,), jnp.int32),
        pltpu.VMEM((_PER_W,), jnp.float32),
        pltpu.SemaphoreType.DMA,
    ],
)
def _gather_kernel(times_hbm, perm_hbm, out_hbm, idx_v, vals_v, sem):
    wid = lax.axis_index("s") * _NC + lax.axis_index("c")
    base = wid * _PER_W
    pltpu.sync_copy(perm_hbm.at[pl.ds(base, _PER_W)], idx_v)
    copies = [
        pltpu.async_copy(
            times_hbm.at[idx_v.at[pl.ds(j * _CHUNK, _CHUNK)]],
            vals_v.at[pl.ds(j * _CHUNK, _CHUNK)],
            sem,
        )
        for j in range(_N_CHUNK)
    ]
    for c in copies:
        c.wait()
    pltpu.sync_copy(vals_v, out_hbm.at[pl.ds(base, _PER_W)])


def kernel(times, perm):
    times_1d = lax.reshape(times, (NT,), dimensions=(1, 0))
    out = _gather_kernel(times_1d, perm.astype(jnp.int32))
    return out.reshape(BATCH, 1)


# speedup vs baseline: 1.0620x; 1.0620x over previous
import functools

import jax
import jax.numpy as jnp
from jax import lax
from jax.experimental import pallas as pl
from jax.experimental.pallas import tpu as pltpu
from jax.experimental.pallas import tpu_sc as plsc

NT = 1000000
BATCH = 16384

_info = plsc.get_sparse_core_info()
_NC, _NS = _info.num_cores, _info.num_subcores
_NW = _NC * _NS
_PER_W = BATCH // _NW
_CHUNK = 128
_N_CHUNK = _PER_W // _CHUNK

_mesh = plsc.VectorSubcoreMesh(core_axis_name="c", subcore_axis_name="s")


@functools.partial(
    pl.kernel,
    out_type=jax.ShapeDtypeStruct((1, BATCH), jnp.float32),
    mesh=_mesh,
    compiler_params=pltpu.CompilerParams(use_tc_tiling_on_sc=False),
    scratch_types=[
        pltpu.VMEM((_N_CHUNK, _CHUNK), jnp.int32),
        pltpu.VMEM((_N_CHUNK, _CHUNK), jnp.float32),
        pltpu.SemaphoreType.DMA,
    ],
)
def _gather_kernel(times_t_hbm, perm_hbm, out_hbm, idx_v, vals_v, sem):
    wid = lax.axis_index("s") * _NC + lax.axis_index("c")
    base = wid * _PER_W
    for j in range(_N_CHUNK):
        pltpu.sync_copy(perm_hbm.at[pl.ds(base + j * _CHUNK, _CHUNK)], idx_v.at[j])
    flat = times_t_hbm.at[0]
    copies = [
        pltpu.async_copy(flat.at[idx_v.at[j]], vals_v.at[j], sem)
        for j in range(_N_CHUNK)
    ]
    for c in copies:
        c.wait()
    flat_out = out_hbm.at[0]
    for j in range(_N_CHUNK):
        pltpu.sync_copy(
            vals_v.at[j], flat_out.at[pl.ds(base + j * _CHUNK, _CHUNK)]
        )


def kernel(times, perm):
    return _gather_kernel(times.T, perm.astype(jnp.int32)).T
